# baseline (speedup 1.0000x reference)
import jax
import jax.numpy as jnp
from jax import lax
from jax.experimental import pallas as pl
from jax.experimental.pallas import tpu as pltpu
from jax.experimental.pallas import tpu_sc as plsc

N = 10000
E = 320000
EL = 65536
D = 128
NC, NS = 2, 16
NW = NC * NS
C = 128

N_PAD = 10240
ROWS_PER_TILE = N_PAD // NS
DEG_PAD = 10240
DEG_PER_TILE = DEG_PAD // NS

E_PER_TILE = E // NW
FULL_CHUNKS = E_PER_TILE // C
TAIL = E_PER_TILE - FULL_CHUNKS * C

_MESH = plsc.VectorSubcoreMesh(core_axis_name="c", subcore_axis_name="s")
_ZROWS = 128


def _make_agg(compute_deg):
    scratch = [
        pltpu.VMEM((E_PER_TILE,), jnp.int32),
        pltpu.VMEM((C, D), jnp.float32),
        pltpu.VMEM((C, D), jnp.float32),
        pltpu.VMEM((C,), jnp.int32),
        pltpu.VMEM((C,), jnp.int32),
        pltpu.VMEM((TAIL,), jnp.int32),
        pltpu.VMEM_SHARED((N_PAD, D), jnp.float32),
        pltpu.SemaphoreType.DMA,
        pltpu.SemaphoreType.DMA,
        pltpu.SemaphoreType.DMA,
        pltpu.SemaphoreType.DMA,
    ]
    if compute_deg:
        scratch += [
            pltpu.VMEM((C,), jnp.float32),
            pltpu.VMEM_SHARED((DEG_PAD,), jnp.float32),
        ]
        out_type = (
            jax.ShapeDtypeStruct((NC, N_PAD, D), jnp.float32),
            jax.ShapeDtypeStruct((NC, DEG_PAD), jnp.float32),
        )
    else:
        out_type = jax.ShapeDtypeStruct((NC, N_PAD, D), jnp.float32)

    def body(h, src, dst, zrows, *rest):
        if compute_deg:
            (zdeg, ones_h, out_agg, out_deg, src_all, rows0, rows1, dstbuf0,
             dstbuf1, tail_dst, agg_sh, gsem0, gsem1, dsem0, dsem1,
             ones_v, deg_sh) = rest
        else:
            (out_agg, src_all, rows0, rows1, dstbuf0, dstbuf1, tail_dst,
             agg_sh, gsem0, gsem1, dsem0, dsem1) = rest
        c = lax.axis_index("c")
        s = lax.axis_index("s")
        wid = s * NC + c
        base = wid * E_PER_TILE

        pltpu.sync_copy(src.at[pl.ds(base, E_PER_TILE)], src_all)
        pltpu.sync_copy(zrows, agg_sh.at[pl.ds(s * ROWS_PER_TILE, ROWS_PER_TILE)])
        if compute_deg:
            pltpu.sync_copy(zdeg, deg_sh.at[pl.ds(s * DEG_PER_TILE, DEG_PER_TILE)])
            pltpu.sync_copy(ones_h, ones_v)
        plsc.subcore_barrier()

        def issue_front(rows, dbuf, gsem, dsem, k):
            pltpu.async_copy(dst.at[pl.ds(base + k * C, C)], dbuf, dsem)
            pltpu.async_copy(h.at[src_all.at[pl.ds(k * C, C)]], rows, gsem)

        def wait_front(rows, dbuf, gsem, dsem):
            pltpu.make_async_copy(dst.at[pl.ds(base, C)], dbuf, dsem).wait()
            pltpu.make_async_copy(h.at[src_all.at[pl.ds(0, C)]], rows, gsem).wait()

        def do_scatter(rows, dbuf):
            pltpu.sync_copy(rows, agg_sh.at[dbuf], add=True)
            if compute_deg:
                pltpu.sync_copy(ones_v, deg_sh.at[dbuf], add=True)

        issue_front(rows0, dstbuf0, gsem0, dsem0, 0)
        issue_front(rows1, dstbuf1, gsem1, dsem1, 1)

        def pair(i, _):
            wait_front(rows0, dstbuf0, gsem0, dsem0)
            do_scatter(rows0, dstbuf0)

            @pl.when(i < FULL_CHUNKS // 2 - 1)
            def _():
                issue_front(rows0, dstbuf0, gsem0, dsem0, 2 * i + 2)
            wait_front(rows1, dstbuf1, gsem1, dsem1)
            do_scatter(rows1, dstbuf1)

            @pl.when(i < FULL_CHUNKS // 2 - 1)
            def _():
                issue_front(rows1, dstbuf1, gsem1, dsem1, 2 * i + 3)
            return 0
        lax.fori_loop(0, FULL_CHUNKS // 2, pair, 0)

        pltpu.sync_copy(dst.at[pl.ds(base + FULL_CHUNKS * C, TAIL)], tail_dst)
        pltpu.async_copy(
            h.at[src_all.at[pl.ds(FULL_CHUNKS * C, TAIL)]],
            rows0.at[pl.ds(0, TAIL)], gsem0).wait()
        pltpu.sync_copy(rows0.at[pl.ds(0, TAIL)], agg_sh.at[tail_dst], add=True)
        if compute_deg:
            pltpu.sync_copy(ones_v.at[pl.ds(0, TAIL)], deg_sh.at[tail_dst], add=True)

        plsc.subcore_barrier()

        pltpu.sync_copy(
            agg_sh.at[pl.ds(s * ROWS_PER_TILE, ROWS_PER_TILE)],
            out_agg.at[c, pl.ds(s * ROWS_PER_TILE, ROWS_PER_TILE)])
        if compute_deg:
            pltpu.sync_copy(
                deg_sh.at[pl.ds(s * DEG_PER_TILE, DEG_PER_TILE)],
                out_deg.at[c, pl.ds(s * DEG_PER_TILE, DEG_PER_TILE)])

    return pl.kernel(body, out_type=out_type, mesh=_MESH, scratch_types=scratch)


_agg_with_deg = _make_agg(True)
_agg = _make_agg(False)

_G_TOTAL = 2 * EL
_G_PER_TILE = _G_TOTAL // NW
_G_CHUNKS = _G_PER_TILE // C


def _gather_body(h, idx, out, idx_all, rows0, rows1, sem0, sem1):
    c = lax.axis_index("c")
    s = lax.axis_index("s")
    wid = s * NC + c
    base = wid * _G_PER_TILE
    pltpu.sync_copy(idx.at[pl.ds(base, _G_PER_TILE)], idx_all)

    def issue(rows, sem, k):
        pltpu.async_copy(h.at[idx_all.at[pl.ds(k * C, C)]], rows, sem)

    def wait_g(rows, sem):
        pltpu.make_async_copy(h.at[idx_all.at[pl.ds(0, C)]], rows, sem).wait()

    def issue_w(rows, wsem, k):
        pltpu.async_copy(rows, out.at[pl.ds(base + k * C, C)], wsem)

    def wait_w(rows, wsem):
        pltpu.make_async_copy(rows, out.at[pl.ds(base, C)], wsem).wait()

    issue(rows0, sem0, 0)
    issue(rows1, sem1, 1)

    def pair(i, _):
        wait_g(rows0, sem0)
        pltpu.sync_copy(rows0, out.at[pl.ds(base + 2 * i * C, C)])

        @pl.when(i < _G_CHUNKS // 2 - 1)
        def _():
            issue(rows0, sem0, 2 * i + 2)
        wait_g(rows1, sem1)
        pltpu.sync_copy(rows1, out.at[pl.ds(base + (2 * i + 1) * C, C)])

        @pl.when(i < _G_CHUNKS // 2 - 1)
        def _():
            issue(rows1, sem1, 2 * i + 3)
        return 0
    lax.fori_loop(0, _G_CHUNKS // 2, pair, 0)


_gather = pl.kernel(
    _gather_body,
    out_type=jax.ShapeDtypeStruct((_G_TOTAL, D), jnp.float32),
    mesh=_MESH,
    scratch_types=[
        pltpu.VMEM((_G_PER_TILE,), jnp.int32),
        pltpu.VMEM((C, D), jnp.float32),
        pltpu.VMEM((C, D), jnp.float32),
        pltpu.SemaphoreType.DMA,
        pltpu.SemaphoreType.DMA,
    ],
)


def _bdot(a, b):
    return jnp.dot(a.astype(jnp.bfloat16), b.astype(jnp.bfloat16),
                   preferred_element_type=jnp.float32)


def _pre_body(x_ref, w_ref, b_ref, g_ref, bb_ref, a_ref, o_ref):
    h = _bdot(x_ref[...], w_ref[...])
    h = h + b_ref[...]
    mu = jnp.mean(h, axis=0, keepdims=True)
    var = jnp.mean((h - mu) * (h - mu), axis=0, keepdims=True)
    hn = (h - mu) * lax.rsqrt(var + 1e-5) * g_ref[...] + bb_ref[...]
    o_ref[...] = jnp.where(hn >= 0.0, hn, a_ref[0, 0] * hn)


def _pre(x, W_in, b_in, g, bb, a):
    return pl.pallas_call(
        _pre_body,
        out_shape=jax.ShapeDtypeStruct((N, D), jnp.float32),
    )(x, W_in, b_in, g, bb, a)


_LB = 1000


def _layer_body(aggp_ref, degp_ref, h_ref, wc_ref, bc_ref, wih_ref, bih_ref,
                whh_ref, bhh_ref, o_ref):
    d = degp_ref[0] + degp_ref[1]
    invd = 1.0 / jnp.maximum(d, 1.0)
    agg = (aggp_ref[0] + aggp_ref[1]) * invd
    m = _bdot(agg, wc_ref[0]) + bc_ref[0]
    gi = _bdot(m, wih_ref[0]) + bih_ref[0]
    hprev = h_ref[0]
    gh = _bdot(hprev, whh_ref[0]) + bhh_ref[0]
    r = jax.nn.sigmoid(gi[:, :D] + gh[:, :D])
    z = jax.nn.sigmoid(gi[:, D:2 * D] + gh[:, D:2 * D])
    n = jnp.tanh(gi[:, 2 * D:] + r * gh[:, 2 * D:])
    o_ref[...] = (1.0 - z) * n + z * hprev


def _layer(aggp, degp, H_list, W_conv, b_conv, W_ih, b_ih, W_hh, b_hh, layer):
    grid = (N // _LB,)
    return pl.pallas_call(
        _layer_body,
        grid=grid,
        in_specs=[
            pl.BlockSpec((NC, _LB, D), lambda i: (0, i, 0)),
            pl.BlockSpec((NC, _LB, 1), lambda i: (0, i, 0)),
            pl.BlockSpec((1, _LB, D), lambda i: (layer, i, 0)),
            pl.BlockSpec((1, D, D), lambda i: (layer, 0, 0)),
            pl.BlockSpec((1, 1, D), lambda i: (layer, 0, 0)),
            pl.BlockSpec((1, D, 3 * D), lambda i: (layer, 0, 0)),
            pl.BlockSpec((1, 1, 3 * D), lambda i: (layer, 0, 0)),
            pl.BlockSpec((1, D, 3 * D), lambda i: (layer, 0, 0)),
            pl.BlockSpec((1, 1, 3 * D), lambda i: (layer, 0, 0)),
        ],
        out_specs=pl.BlockSpec((_LB, D), lambda i: (i, 0)),
        out_shape=jax.ShapeDtypeStruct((N, D), jnp.float32),
    )(aggp, degp, H_list, W_conv, b_conv.reshape(2, 1, D),
      W_ih, b_ih.reshape(2, 1, 3 * D), W_hh, b_hh.reshape(2, 1, 3 * D))


_DB = 8192


def _dec_body(hs_ref, hd_ref, w1a_ref, w1b_ref, b1_ref, w2_ref, b2_ref, o_ref):
    e = (_bdot(hs_ref[...], w1a_ref[...]) + _bdot(hd_ref[...], w1b_ref[...])
         + b1_ref[...])
    e = jnp.maximum(e, 0.0)
    o_ref[...] = _bdot(e, w2_ref[...]) + b2_ref[...]


def _dec(gath, W_d1, b1, W2, b2):
    grid = (EL // _DB,)
    return pl.pallas_call(
        _dec_body,
        grid=grid,
        in_specs=[
            pl.BlockSpec((_DB, D), lambda i: (i, 0)),
            pl.BlockSpec((_DB, D), lambda i: (i + EL // _DB, 0)),
            pl.BlockSpec((D, D), lambda i: (0, 0)),
            pl.BlockSpec((D, D), lambda i: (1, 0)),
            pl.BlockSpec((1, D), lambda i: (0, 0)),
            pl.BlockSpec((D, 1), lambda i: (0, 0)),
            pl.BlockSpec((1, 1), lambda i: (0, 0)),
        ],
        out_specs=pl.BlockSpec((_DB, 1), lambda i: (i, 0)),
        out_shape=jax.ShapeDtypeStruct((EL, 1), jnp.float32),
    )(gath, gath, W_d1, W_d1, b1, W2, b2)


def kernel(x, edge_index, edge_label_index, H_list, W_in, b_in, bn_gamma,
           bn_beta, prelu_a, W_conv, b_conv, W_ih, W_hh, b_ih, b_hh, W_d1,
           b_d1, W_d2, b_d2):
    src = edge_index[0]
    dst = edge_index[1]
    h = _pre(x, W_in, b_in.reshape(1, D), bn_gamma.reshape(1, D),
             bn_beta.reshape(1, D), prelu_a.reshape(1, 1))

    zrows = jnp.zeros((ROWS_PER_TILE, D), jnp.float32)
    zdeg = jnp.zeros((DEG_PER_TILE,), jnp.float32)
    ones_h = jnp.ones((C,), jnp.float32)

    aggp, degp = _agg_with_deg(h, src, dst, zrows, zdeg, ones_h)
    degp_r = degp.reshape(NC, DEG_PAD, 1)

    h = _layer(aggp, degp_r, H_list, W_conv, b_conv, W_ih, b_ih, W_hh,
               b_hh, 0)

    aggp2 = _agg(h, src, dst, zrows)
    h = _layer(aggp2, degp_r, H_list, W_conv, b_conv, W_ih, b_ih, W_hh,
               b_hh, 1)

    gath = _gather(h, edge_label_index.reshape(_G_TOTAL))

    return _dec(gath, W_d1, b_d1.reshape(1, D), W_d2, b_d2.reshape(1, 1))


# speedup vs baseline: 1.0292x; 1.0292x over previous
import jax
import jax.numpy as jnp
from jax import lax
from jax.experimental import pallas as pl
from jax.experimental.pallas import tpu as pltpu
from jax.experimental.pallas import tpu_sc as plsc

N = 10000
E = 320000
EL = 65536
D = 128
NC, NS = 2, 16
NW = NC * NS
C = 128

N_PAD = 10240
ROWS_PER_TILE = N_PAD // NS
DEG_PAD = 10240
DEG_PER_TILE = DEG_PAD // NS

E_PER_TILE = E // NW
FULL_CHUNKS = E_PER_TILE // C
TAIL = E_PER_TILE - FULL_CHUNKS * C

_MESH = plsc.VectorSubcoreMesh(core_axis_name="c", subcore_axis_name="s")
_ZROWS = 128


def _make_agg(compute_deg):
    scratch = [
        pltpu.VMEM((E_PER_TILE,), jnp.int32),
        pltpu.VMEM((C, D), jnp.float32),
        pltpu.VMEM((C, D), jnp.float32),
        pltpu.VMEM((C,), jnp.int32),
        pltpu.VMEM((C,), jnp.int32),
        pltpu.VMEM((TAIL,), jnp.int32),
        pltpu.VMEM_SHARED((N_PAD, D), jnp.float32),
        pltpu.SemaphoreType.DMA,
        pltpu.SemaphoreType.DMA,
        pltpu.SemaphoreType.DMA,
        pltpu.SemaphoreType.DMA,
    ]
    if compute_deg:
        scratch += [
            pltpu.VMEM((C,), jnp.float32),
            pltpu.VMEM_SHARED((DEG_PAD,), jnp.float32),
        ]
        out_type = (
            jax.ShapeDtypeStruct((NC, N_PAD, D), jnp.float32),
            jax.ShapeDtypeStruct((NC, DEG_PAD), jnp.float32),
        )
    else:
        out_type = jax.ShapeDtypeStruct((NC, N_PAD, D), jnp.float32)

    def body(h, ei_flat, zrows, *rest):
        if compute_deg:
            (zdeg, ones_h, out_agg, out_deg, src_all, rows0, rows1, dstbuf0,
             dstbuf1, tail_dst, agg_sh, gsem0, gsem1, dsem0, dsem1,
             ones_v, deg_sh) = rest
        else:
            (out_agg, src_all, rows0, rows1, dstbuf0, dstbuf1, tail_dst,
             agg_sh, gsem0, gsem1, dsem0, dsem1) = rest
        c = lax.axis_index("c")
        s = lax.axis_index("s")
        wid = s * NC + c
        base = wid * E_PER_TILE

        pltpu.sync_copy(ei_flat.at[pl.ds(base, E_PER_TILE)], src_all)
        pltpu.sync_copy(zrows, agg_sh.at[pl.ds(s * ROWS_PER_TILE, ROWS_PER_TILE)])
        if compute_deg:
            pltpu.sync_copy(zdeg, deg_sh.at[pl.ds(s * DEG_PER_TILE, DEG_PER_TILE)])
            pltpu.sync_copy(ones_h, ones_v)
        plsc.subcore_barrier()

        def issue_front(rows, dbuf, gsem, dsem, k):
            pltpu.async_copy(ei_flat.at[pl.ds(E + base + k * C, C)], dbuf, dsem)
            pltpu.async_copy(h.at[src_all.at[pl.ds(k * C, C)]], rows, gsem)

        def wait_front(rows, dbuf, gsem, dsem):
            pltpu.make_async_copy(ei_flat.at[pl.ds(E, C)], dbuf, dsem).wait()
            pltpu.make_async_copy(h.at[src_all.at[pl.ds(0, C)]], rows, gsem).wait()

        def do_scatter(rows, dbuf):
            pltpu.sync_copy(rows, agg_sh.at[dbuf], add=True)
            if compute_deg:
                pltpu.sync_copy(ones_v, deg_sh.at[dbuf], add=True)

        issue_front(rows0, dstbuf0, gsem0, dsem0, 0)
        issue_front(rows1, dstbuf1, gsem1, dsem1, 1)

        def pair(i, _):
            wait_front(rows0, dstbuf0, gsem0, dsem0)
            do_scatter(rows0, dstbuf0)

            @pl.when(i < FULL_CHUNKS // 2 - 1)
            def _():
                issue_front(rows0, dstbuf0, gsem0, dsem0, 2 * i + 2)
            wait_front(rows1, dstbuf1, gsem1, dsem1)
            do_scatter(rows1, dstbuf1)

            @pl.when(i < FULL_CHUNKS // 2 - 1)
            def _():
                issue_front(rows1, dstbuf1, gsem1, dsem1, 2 * i + 3)
            return 0
        lax.fori_loop(0, FULL_CHUNKS // 2, pair, 0)

        pltpu.sync_copy(
            ei_flat.at[pl.ds(E + base + FULL_CHUNKS * C, TAIL)], tail_dst)
        pltpu.async_copy(
            h.at[src_all.at[pl.ds(FULL_CHUNKS * C, TAIL)]],
            rows0.at[pl.ds(0, TAIL)], gsem0).wait()
        pltpu.sync_copy(rows0.at[pl.ds(0, TAIL)], agg_sh.at[tail_dst], add=True)
        if compute_deg:
            pltpu.sync_copy(ones_v.at[pl.ds(0, TAIL)], deg_sh.at[tail_dst], add=True)

        plsc.subcore_barrier()

        pltpu.sync_copy(
            agg_sh.at[pl.ds(s * ROWS_PER_TILE, ROWS_PER_TILE)],
            out_agg.at[c, pl.ds(s * ROWS_PER_TILE, ROWS_PER_TILE)])
        if compute_deg:
            pltpu.sync_copy(
                deg_sh.at[pl.ds(s * DEG_PER_TILE, DEG_PER_TILE)],
                out_deg.at[c, pl.ds(s * DEG_PER_TILE, DEG_PER_TILE)])

    return pl.kernel(body, out_type=out_type, mesh=_MESH, scratch_types=scratch)


_agg_with_deg = _make_agg(True)
_agg = _make_agg(False)

_G_TOTAL = 2 * EL
_G_PER_TILE = _G_TOTAL // NW
_G_CHUNKS = _G_PER_TILE // C


def _gather_body(h, idx, out, idx_all, rows0, rows1, sem0, sem1):
    c = lax.axis_index("c")
    s = lax.axis_index("s")
    wid = s * NC + c
    base = wid * _G_PER_TILE
    pltpu.sync_copy(idx.at[pl.ds(base, _G_PER_TILE)], idx_all)

    def issue(rows, sem, k):
        pltpu.async_copy(h.at[idx_all.at[pl.ds(k * C, C)]], rows, sem)

    def wait_g(rows, sem):
        pltpu.make_async_copy(h.at[idx_all.at[pl.ds(0, C)]], rows, sem).wait()

    def issue_w(rows, wsem, k):
        pltpu.async_copy(rows, out.at[pl.ds(base + k * C, C)], wsem)

    def wait_w(rows, wsem):
        pltpu.make_async_copy(rows, out.at[pl.ds(base, C)], wsem).wait()

    issue(rows0, sem0, 0)
    issue(rows1, sem1, 1)

    def pair(i, _):
        wait_g(rows0, sem0)
        pltpu.sync_copy(rows0, out.at[pl.ds(base + 2 * i * C, C)])

        @pl.when(i < _G_CHUNKS // 2 - 1)
        def _():
            issue(rows0, sem0, 2 * i + 2)
        wait_g(rows1, sem1)
        pltpu.sync_copy(rows1, out.at[pl.ds(base + (2 * i + 1) * C, C)])

        @pl.when(i < _G_CHUNKS // 2 - 1)
        def _():
            issue(rows1, sem1, 2 * i + 3)
        return 0
    lax.fori_loop(0, _G_CHUNKS // 2, pair, 0)


_gather = pl.kernel(
    _gather_body,
    out_type=jax.ShapeDtypeStruct((_G_TOTAL, D), jnp.float32),
    mesh=_MESH,
    scratch_types=[
        pltpu.VMEM((_G_PER_TILE,), jnp.int32),
        pltpu.VMEM((C, D), jnp.float32),
        pltpu.VMEM((C, D), jnp.float32),
        pltpu.SemaphoreType.DMA,
        pltpu.SemaphoreType.DMA,
    ],
)


def _bdot(a, b):
    return jnp.dot(a.astype(jnp.bfloat16), b.astype(jnp.bfloat16),
                   preferred_element_type=jnp.float32)


def _pre_body(x_ref, w_ref, b_ref, g_ref, bb_ref, a_ref, o_ref):
    h = _bdot(x_ref[...], w_ref[...])
    h = h + b_ref[...]
    mu = jnp.mean(h, axis=0, keepdims=True)
    var = jnp.mean((h - mu) * (h - mu), axis=0, keepdims=True)
    hn = (h - mu) * lax.rsqrt(var + 1e-5) * g_ref[...] + bb_ref[...]
    o_ref[...] = jnp.where(hn >= 0.0, hn, a_ref[0, 0] * hn)


def _pre(x, W_in, b_in, g, bb, a):
    return pl.pallas_call(
        _pre_body,
        out_shape=jax.ShapeDtypeStruct((N, D), jnp.float32),
    )(x, W_in, b_in, g, bb, a)


_LB = 1000


def _layer_body(aggp_ref, degp_ref, h_ref, wc_ref, bc_ref, wih_ref, bih_ref,
                whh_ref, bhh_ref, o_ref):
    d = degp_ref[0] + degp_ref[1]
    invd = 1.0 / jnp.maximum(d, 1.0)
    agg = (aggp_ref[0] + aggp_ref[1]) * invd
    m = _bdot(agg, wc_ref[0]) + bc_ref[0]
    gi = _bdot(m, wih_ref[0]) + bih_ref[0]
    hprev = h_ref[0]
    gh = _bdot(hprev, whh_ref[0]) + bhh_ref[0]
    r = jax.nn.sigmoid(gi[:, :D] + gh[:, :D])
    z = jax.nn.sigmoid(gi[:, D:2 * D] + gh[:, D:2 * D])
    n = jnp.tanh(gi[:, 2 * D:] + r * gh[:, 2 * D:])
    o_ref[...] = (1.0 - z) * n + z * hprev


def _layer(aggp, degp, H_list, W_conv, b_conv, W_ih, b_ih, W_hh, b_hh, layer):
    grid = (N // _LB,)
    return pl.pallas_call(
        _layer_body,
        grid=grid,
        in_specs=[
            pl.BlockSpec((NC, _LB, D), lambda i: (0, i, 0)),
            pl.BlockSpec((NC, _LB, 1), lambda i: (0, i, 0)),
            pl.BlockSpec((1, _LB, D), lambda i: (layer, i, 0)),
            pl.BlockSpec((1, D, D), lambda i: (layer, 0, 0)),
            pl.BlockSpec((1, 1, D), lambda i: (layer, 0, 0)),
            pl.BlockSpec((1, D, 3 * D), lambda i: (layer, 0, 0)),
            pl.BlockSpec((1, 1, 3 * D), lambda i: (layer, 0, 0)),
            pl.BlockSpec((1, D, 3 * D), lambda i: (layer, 0, 0)),
            pl.BlockSpec((1, 1, 3 * D), lambda i: (layer, 0, 0)),
        ],
        out_specs=pl.BlockSpec((_LB, D), lambda i: (i, 0)),
        out_shape=jax.ShapeDtypeStruct((N, D), jnp.float32),
    )(aggp, degp, H_list, W_conv, b_conv.reshape(2, 1, D),
      W_ih, b_ih.reshape(2, 1, 3 * D), W_hh, b_hh.reshape(2, 1, 3 * D))


_DB = 8192


def _dec_body(hs_ref, hd_ref, w1a_ref, w1b_ref, b1_ref, w2_ref, b2_ref, o_ref):
    e = (_bdot(hs_ref[...], w1a_ref[...]) + _bdot(hd_ref[...], w1b_ref[...])
         + b1_ref[...])
    e = jnp.maximum(e, 0.0)
    o_ref[...] = _bdot(e, w2_ref[...]) + b2_ref[...]


def _dec(gath, W_d1, b1, W2, b2):
    grid = (EL // _DB,)
    return pl.pallas_call(
        _dec_body,
        grid=grid,
        in_specs=[
            pl.BlockSpec((_DB, D), lambda i: (i, 0)),
            pl.BlockSpec((_DB, D), lambda i: (i + EL // _DB, 0)),
            pl.BlockSpec((D, D), lambda i: (0, 0)),
            pl.BlockSpec((D, D), lambda i: (1, 0)),
            pl.BlockSpec((1, D), lambda i: (0, 0)),
            pl.BlockSpec((D, 1), lambda i: (0, 0)),
            pl.BlockSpec((1, 1), lambda i: (0, 0)),
        ],
        out_specs=pl.BlockSpec((_DB, 1), lambda i: (i, 0)),
        out_shape=jax.ShapeDtypeStruct((EL, 1), jnp.float32),
    )(gath, gath, W_d1, W_d1, b1, W2, b2)


def kernel(x, edge_index, edge_label_index, H_list, W_in, b_in, bn_gamma,
           bn_beta, prelu_a, W_conv, b_conv, W_ih, W_hh, b_ih, b_hh, W_d1,
           b_d1, W_d2, b_d2):
    ei_flat = edge_index.reshape(2 * E)
    h = _pre(x, W_in, b_in.reshape(1, D), bn_gamma.reshape(1, D),
             bn_beta.reshape(1, D), prelu_a.reshape(1, 1))

    zrows = jnp.zeros((ROWS_PER_TILE, D), jnp.float32)
    zdeg = jnp.zeros((DEG_PER_TILE,), jnp.float32)
    ones_h = jnp.ones((C,), jnp.float32)

    aggp, degp = _agg_with_deg(h, ei_flat, zrows, zdeg, ones_h)
    degp_r = degp.reshape(NC, DEG_PAD, 1)

    h = _layer(aggp, degp_r, H_list, W_conv, b_conv, W_ih, b_ih, W_hh,
               b_hh, 0)

    aggp2 = _agg(h, ei_flat, zrows)
    h = _layer(aggp2, degp_r, H_list, W_conv, b_conv, W_ih, b_ih, W_hh,
               b_hh, 1)

    gath = _gather(h, edge_label_index.reshape(_G_TOTAL))

    return _dec(gath, W_d1, b_d1.reshape(1, D), W_d2, b_d2.reshape(1, 1))
